# baseline (speedup 1.0000x reference)
import functools

import jax
import jax.numpy as jnp
import numpy as _np
from jax.experimental import pallas as pl
from jax.experimental.pallas import tpu as pltpu

_BB = 16
_N = 50
_L = 3
_K = 256


def _bdot(a, b, dims):
    return jax.lax.dot_general(a.astype(jnp.bfloat16), b, (dims, ((), ())),
                               preferred_element_type=jnp.float32)


def _fused_body(x_ref, w0_ref, b0_ref, w1_ref, b1_ref, w2_ref, b2_ref,
                w3_ref, b3_ref, cb_ref, cc_ref, out_ref,
                w0s, w1s, w2s, w3s, cb1s, cb2s, cb3s, xb, xsem):
    bf = jnp.bfloat16
    f32 = jnp.float32
    i = pl.program_id(0)
    nb = pl.num_programs(0)

    @pl.when(i == 0)
    def _prep():
        pltpu.make_async_copy(x_ref.at[pl.ds(0, _BB)], xb.at[0],
                              xsem.at[0]).start()
        w0s[...] = w0_ref[...].astype(bf)
        w1s[...] = w1_ref[...].astype(bf)
        w2s[...] = w2_ref[...].astype(bf)
        w3s[...] = w3_ref[...].astype(bf)
        cb = cb_ref[...]
        c1 = cb.astype(bf)
        r1 = cb - c1.astype(f32)
        c2 = r1.astype(bf)
        cb1s[...] = c1
        cb2s[...] = c2
        cb3s[...] = (r1 - c2.astype(f32)).astype(bf)

    @pl.when(i + 1 < nb)
    def _prefetch():
        nxt = jax.lax.rem(i + 1, 2)
        pltpu.make_async_copy(x_ref.at[pl.ds((i + 1) * _BB, _BB)],
                              xb.at[nxt], xsem.at[nxt]).start()

    slot = jax.lax.rem(i, 2)
    pltpu.make_async_copy(x_ref.at[pl.ds(i * _BB, _BB)], xb.at[slot],
                          xsem.at[slot]).wait()

    rows = _BB * _N
    h = xb[slot].reshape(rows, x_ref.shape[-1])
    h = jnp.maximum(_bdot(h, w0s[...], ((1,), (0,))) + b0_ref[...], 0.0)
    h = jnp.maximum(_bdot(h, w1s[...], ((1,), (0,))) + b1_ref[...], 0.0)
    h = jnp.maximum(_bdot(h, w2s[...], ((1,), (0,))) + b2_ref[...], 0.0)
    r = _bdot(h, w3s[...], ((1,), (0,))) + b3_ref[...]

    cols = []
    for l in range(_L):
        rr = jnp.sum(r * r, axis=-1, keepdims=True)
        rc = _bdot(r, cb1s[l], ((1,), (1,)))
        d = rr - 2.0 * rc + cc_ref[l][None, :]
        idx = jnp.argmin(d, axis=-1)
        cols.append(idx)
        if l < _L - 1:
            oh = (jax.lax.broadcasted_iota(jnp.int32, (rows, _K), 1)
                  == idx[:, None]).astype(bf)
            sel = lambda cbp: jax.lax.dot_general(
                oh, cbp, ((((1,), (0,))), ((), ())),
                preferred_element_type=f32)
            r = r - (sel(cb1s[l]) + sel(cb2s[l]) + sel(cb3s[l]))

    riota = jax.lax.broadcasted_iota(jnp.int32, (rows, _N * _L), 0)
    jiota = jax.lax.broadcasted_iota(jnp.int32, (rows, _N * _L), 1)
    base = _L * (riota % _N)
    b_acc = jnp.zeros((rows, _N * _L), jnp.int32)
    for l in range(_L):
        m = 1 - jnp.minimum(jnp.abs(jiota - base - l), 1)
        b_acc += m * cols[l][:, None]
    b_mat = b_acc.astype(bf)
    pi = jax.lax.broadcasted_iota(jnp.int32, (_BB, rows), 0)
    ri = jax.lax.broadcasted_iota(jnp.int32, (_BB, rows), 1)
    u_mat = (1 - jnp.minimum(jnp.abs(ri // _N - pi), 1)).astype(bf)
    out = jax.lax.dot_general(u_mat, b_mat, ((((1,), (0,))), ((), ())),
                              preferred_element_type=f32)
    out_ref[...] = out.astype(jnp.int32)


@functools.partial(jax.jit, static_argnames=())
def kernel(x, ids, ids_fut, user_ids, seq_mask, codebooks,
           W0, b0, W1, b1, W2, b2, W3, b3):
    Bb, Nn = ids.shape
    bf = jnp.bfloat16
    cc = jnp.sum(codebooks * codebooks, axis=-1)

    full = lambda *shape: pl.BlockSpec(shape, lambda i: (0,) * len(shape))
    sem_ids = pl.pallas_call(
        _fused_body,
        grid=(Bb // _BB,),
        in_specs=[
            pl.BlockSpec(memory_space=pltpu.MemorySpace.HBM),
            full(*W0.shape), full(1, b0.shape[0]),
            full(*W1.shape), full(1, b1.shape[0]),
            full(*W2.shape), full(1, b2.shape[0]),
            full(*W3.shape), full(1, b3.shape[0]),
            full(*codebooks.shape), full(*cc.shape),
        ],
        out_specs=pl.BlockSpec((_BB, Nn * _L), lambda i: (i, 0)),
        out_shape=jax.ShapeDtypeStruct((Bb, Nn * _L), jnp.int32),
        scratch_shapes=[
            pltpu.VMEM(W0.shape, bf), pltpu.VMEM(W1.shape, bf),
            pltpu.VMEM(W2.shape, bf), pltpu.VMEM(W3.shape, bf),
            pltpu.VMEM(codebooks.shape, bf), pltpu.VMEM(codebooks.shape, bf),
            pltpu.VMEM(codebooks.shape, bf),
            pltpu.VMEM((2, _BB, Nn, x.shape[-1]), jnp.float32),
            pltpu.SemaphoreType.DMA((2,)),
        ],
    )(x, W0, b0[None, :], W1, b1[None, :], W2, b2[None, :], W3, b3[None, :],
      codebooks, cc)

    token_type_ids = jnp.asarray(_np.tile(_np.arange(_L), (Bb, Nn)),
                                 dtype=jnp.int32)
    token_type_ids_fut = jnp.asarray(_np.tile(_np.arange(_L), (Bb, 1)),
                                     dtype=jnp.int32)
    return (sem_ids, token_type_ids, token_type_ids_fut)


# speedup vs baseline: 1.0209x; 1.0209x over previous
import functools

import jax
import jax.numpy as jnp
import numpy as _np
from jax.experimental import pallas as pl
from jax.experimental.pallas import tpu as pltpu

_BB = 16
_N = 50
_L = 3
_K = 256


def _bdot(a, b, dims):
    return jax.lax.dot_general(a.astype(jnp.bfloat16), b, (dims, ((), ())),
                               preferred_element_type=jnp.float32)


def _fused_body(x_ref, w0_ref, b0_ref, w1_ref, b1_ref, w2_ref, b2_ref,
                w3_ref, b3_ref, cb_ref, cc_ref, out_ref,
                w0s, w1s, w2s, w3s, cb1s, cb2s, cb3s):
    bf = jnp.bfloat16
    f32 = jnp.float32
    i = pl.program_id(0)

    @pl.when(i == 0)
    def _prep():
        w0s[...] = w0_ref[...].astype(bf)
        w1s[...] = w1_ref[...].astype(bf)
        w2s[...] = w2_ref[...].astype(bf)
        w3s[...] = w3_ref[...].astype(bf)
        cb = cb_ref[...]
        c1 = cb.astype(bf)
        r1 = cb - c1.astype(f32)
        c2 = r1.astype(bf)
        cb1s[...] = c1
        cb2s[...] = c2
        cb3s[...] = (r1 - c2.astype(f32)).astype(bf)

    rows = _BB * _N
    h = x_ref[...].reshape(rows, x_ref.shape[-1])
    h = jnp.maximum(_bdot(h, w0s[...], ((1,), (0,))) + b0_ref[...], 0.0)
    h = jnp.maximum(_bdot(h, w1s[...], ((1,), (0,))) + b1_ref[...], 0.0)
    h = jnp.maximum(_bdot(h, w2s[...], ((1,), (0,))) + b2_ref[...], 0.0)
    r = _bdot(h, w3s[...], ((1,), (0,))) + b3_ref[...]

    cols = []
    for l in range(_L):
        rr = jnp.sum(r * r, axis=-1, keepdims=True)
        rc = _bdot(r, cb1s[l], ((1,), (1,)))
        d = rr - 2.0 * rc + cc_ref[l][None, :]
        idx = jnp.argmin(d, axis=-1)
        cols.append(idx)
        if l < _L - 1:
            oh = (jax.lax.broadcasted_iota(jnp.int32, (rows, _K), 1)
                  == idx[:, None]).astype(bf)
            sel = lambda cbp: jax.lax.dot_general(
                oh, cbp, ((((1,), (0,))), ((), ())),
                preferred_element_type=f32)
            r = r - (sel(cb1s[l]) + sel(cb2s[l]) + sel(cb3s[l]))

    riota = jax.lax.broadcasted_iota(jnp.int32, (rows, _N * _L), 0)
    jiota = jax.lax.broadcasted_iota(jnp.int32, (rows, _N * _L), 1)
    base = _L * (riota % _N)
    b_acc = jnp.zeros((rows, _N * _L), jnp.int32)
    for l in range(_L):
        m = 1 - jnp.minimum(jnp.abs(jiota - base - l), 1)
        b_acc += m * cols[l][:, None]
    b_mat = b_acc.astype(bf)
    pi = jax.lax.broadcasted_iota(jnp.int32, (_BB, rows), 0)
    ri = jax.lax.broadcasted_iota(jnp.int32, (_BB, rows), 1)
    u_mat = (1 - jnp.minimum(jnp.abs(ri // _N - pi), 1)).astype(bf)
    out = jax.lax.dot_general(u_mat, b_mat, ((((1,), (0,))), ((), ())),
                              preferred_element_type=f32)
    out_ref[...] = out.astype(jnp.int32)


@functools.partial(jax.jit, static_argnames=())
def kernel(x, ids, ids_fut, user_ids, seq_mask, codebooks,
           W0, b0, W1, b1, W2, b2, W3, b3):
    Bb, Nn = ids.shape
    bf = jnp.bfloat16
    cc = jnp.sum(codebooks * codebooks, axis=-1)

    full = lambda *shape: pl.BlockSpec(shape, lambda i: (0,) * len(shape))
    sem_ids = pl.pallas_call(
        _fused_body,
        grid=(Bb // _BB,),
        in_specs=[
            pl.BlockSpec((_BB, Nn, x.shape[-1]), lambda i: (i, 0, 0)),
            full(*W0.shape), full(1, b0.shape[0]),
            full(*W1.shape), full(1, b1.shape[0]),
            full(*W2.shape), full(1, b2.shape[0]),
            full(*W3.shape), full(1, b3.shape[0]),
            full(*codebooks.shape), full(*cc.shape),
        ],
        out_specs=pl.BlockSpec((_BB, Nn * _L), lambda i: (i, 0)),
        out_shape=jax.ShapeDtypeStruct((Bb, Nn * _L), jnp.int32),
        scratch_shapes=[
            pltpu.VMEM(W0.shape, bf), pltpu.VMEM(W1.shape, bf),
            pltpu.VMEM(W2.shape, bf), pltpu.VMEM(W3.shape, bf),
            pltpu.VMEM(codebooks.shape, bf), pltpu.VMEM(codebooks.shape, bf),
            pltpu.VMEM(codebooks.shape, bf),
        ],
    )(x, W0, b0[None, :], W1, b1[None, :], W2, b2[None, :], W3, b3[None, :],
      codebooks, cc)

    token_type_ids = jnp.asarray(_np.tile(_np.arange(_L), (Bb, Nn)),
                                 dtype=jnp.int32)
    token_type_ids_fut = jnp.asarray(_np.tile(_np.arange(_L), (Bb, 1)),
                                     dtype=jnp.int32)
    return (sem_ids, token_type_ids, token_type_ids_fut)
